# Initial kernel scaffold; baseline (speedup 1.0000x reference)
import jax
import jax.numpy as jnp
from jax.experimental import pallas as pl


def kernel(features, edge_index, norm_A, W1, b1, W2, b2, alpha):
    raise NotImplementedError("write your pallas kernel here")


# speedup vs baseline: 3.9347x; 3.9347x over previous
import functools

import jax
import jax.numpy as jnp
from jax import lax
from jax.experimental import pallas as pl
from jax.experimental.pallas import tpu as pltpu
from jax.experimental.pallas import tpu_sc as plsc

N = 10000
E = 320000
D = 64

NC = 2
NS = 16
LANES = 16
CHUNK = 128
CPT = 79
EPT = CHUNK * CPT
EPAD = EPT * NC * NS
NPAD = 10240
ROWS_PER_TILE = NPAD // NS


def _init_body(f_ref, w1_ref, b1_ref, w2_ref, b2_ref, a0_ref, h0_ref, rst_ref):
    x = jnp.dot(f_ref[...], w1_ref[...], preferred_element_type=jnp.float32)
    x = jnp.maximum(x + b1_ref[...][None, :], 0.0)
    x = jnp.dot(x, w2_ref[...], preferred_element_type=jnp.float32)
    x = x + b2_ref[...][None, :]
    ss = jnp.sum(x * x, axis=0)
    h0 = x / jnp.maximum(jnp.sqrt(ss), 1e-8)[None, :]
    h0_ref[...] = h0
    rst_ref[...] = a0_ref[...][None, :] * h0


def _init_call(features, W1, b1, W2, b2, a0):
    return pl.pallas_call(
        _init_body,
        out_shape=(
            jax.ShapeDtypeStruct((N, D), jnp.float32),
            jax.ShapeDtypeStruct((N, D), jnp.float32),
        ),
    )(features, W1, b1, W2, b2, a0)


def _spmm_body(h_hbm, src_hbm, dst_hbm, w_hbm, out_hbm,
               src_v, dst_v, w_v, rows_v, zero_v, acc_sh, sem):
    cid = lax.axis_index("c")
    sid = lax.axis_index("s")
    tid = cid * NS + sid

    def zrow(i, _):
        for c4 in range(D // LANES):
            zero_v[i, pl.ds(c4 * LANES, LANES)] = jnp.zeros((LANES,), jnp.float32)
        return 0
    lax.fori_loop(0, ROWS_PER_TILE, zrow, 0)
    pltpu.sync_copy(zero_v, acc_sh.at[pl.ds(sid * ROWS_PER_TILE, ROWS_PER_TILE)])
    plsc.subcore_barrier()

    def chunk_body(k, _):
        off = tid * EPT + k * CHUNK
        pltpu.sync_copy(src_hbm.at[pl.ds(off, CHUNK)], src_v)
        pltpu.sync_copy(dst_hbm.at[pl.ds(off, CHUNK)], dst_v)
        pltpu.sync_copy(w_hbm.at[pl.ds(off, CHUNK)], w_v)
        pltpu.async_copy(h_hbm.at[src_v], rows_v, sem).wait()

        def mul_body(j, _):
            wj = plsc.load_gather(w_v, [jnp.full((LANES,), j, jnp.int32)])
            for c4 in range(D // LANES):
                sl = pl.ds(c4 * LANES, LANES)
                rows_v[j, sl] = rows_v[j, sl] * wj
            return 0
        lax.fori_loop(0, CHUNK, mul_body, 0)

        pltpu.sync_copy(rows_v, acc_sh.at[dst_v], add=True)
        return 0

    lax.fori_loop(0, CPT, chunk_body, 0)
    plsc.subcore_barrier()

    r0 = sid * ROWS_PER_TILE
    pltpu.sync_copy(acc_sh.at[pl.ds(r0, ROWS_PER_TILE)],
                    out_hbm.at[cid, pl.ds(r0, ROWS_PER_TILE)])


@functools.partial(
    pl.kernel,
    out_type=jax.ShapeDtypeStruct((NC, NPAD, D), jnp.float32),
    mesh=plsc.VectorSubcoreMesh(core_axis_name="c", subcore_axis_name="s"),
    compiler_params=pltpu.CompilerParams(
        needs_layout_passes=False, use_tc_tiling_on_sc=False),
    scratch_types=[
        pltpu.VMEM((CHUNK,), jnp.int32),
        pltpu.VMEM((CHUNK,), jnp.int32),
        pltpu.VMEM((CHUNK,), jnp.float32),
        pltpu.VMEM((CHUNK, D), jnp.float32),
        pltpu.VMEM((ROWS_PER_TILE, D), jnp.float32),
        pltpu.VMEM_SHARED((NPAD, D), jnp.float32),
        pltpu.SemaphoreType.DMA,
    ],
)
def _spmm_call(h_hbm, src_hbm, dst_hbm, w_hbm, out_hbm,
               src_v, dst_v, w_v, rows_v, zero_v, acc_sh, sem):
    _spmm_body(h_hbm, src_hbm, dst_hbm, w_hbm, out_hbm,
               src_v, dst_v, w_v, rows_v, zero_v, acc_sh, sem)


def _hop_body(p_ref, last_ref, second_ref, acc_ref, ai_ref, h_ref, out_ref):
    r = p_ref[0, :N] + p_ref[1, :N]
    last = last_ref[...]
    second = second_ref[...]
    t1 = jnp.sum(r * last, axis=0)
    r = r - t1[None, :] * last
    t2 = jnp.sum(r * second, axis=0)
    r = r - t2[None, :] * second
    nrm = jnp.sqrt(jnp.sum(r * r, axis=0))
    h = r / jnp.maximum(nrm, 1e-8)[None, :]
    h_ref[...] = h
    out_ref[...] = acc_ref[...] + ai_ref[...][None, :] * h


def _hop_call(p, last, second, acc, ai):
    return pl.pallas_call(
        _hop_body,
        out_shape=(
            jax.ShapeDtypeStruct((N, D), jnp.float32),
            jax.ShapeDtypeStruct((N, D), jnp.float32),
        ),
    )(p, last, second, acc, ai)


def kernel(features, edge_index, norm_A, W1, b1, W2, b2, alpha):
    K = alpha.shape[1] - 1
    pad = EPAD - E
    src = jnp.concatenate([edge_index[0], jnp.zeros((pad,), jnp.int32)])
    dst = jnp.concatenate([edge_index[1], jnp.zeros((pad,), jnp.int32)])
    w = jnp.concatenate([norm_A, jnp.zeros((pad,), jnp.float32)])

    h0, rst = _init_call(features, W1, b1, W2, b2, alpha[:, 0])
    last = h0
    second = jnp.zeros_like(h0)
    for i in range(1, K + 1):
        p = _spmm_call(last, src, dst, w)
        h_new, rst = _hop_call(p, last, second, rst, alpha[:, i])
        second = last
        last = h_new
    return rst
